# baseline (speedup 1.0000x reference)
import functools

import jax
import jax.numpy as jnp
from jax import lax
from jax.experimental import pallas as pl
from jax.experimental.pallas import tpu as pltpu
from jax.experimental.pallas import tpu_sc as plsc

_N_FACTORS = 64
_BATCH = 16384
_NC, _NS, _L = 2, 16, 16
_NW = _NC * _NS
_BPW = _BATCH // _NW
_GROUPS = _BPW // _L
_FEAT_BLK = 2048


def _feat_body(gb_ref, uf_ref, if_ref, wu_ref, wi_ref, out_ref):
    acc = jnp.sum(uf_ref[...] * wu_ref[...], axis=1)
    acc = acc + jnp.sum(if_ref[...] * wi_ref[...], axis=1)
    out_ref[...] = acc + gb_ref[0]


def _feat_call(global_bias, user_features, item_features, w_u, w_i):
    batch, fdim = user_features.shape
    grid = batch // _FEAT_BLK
    return pl.pallas_call(
        _feat_body,
        grid=(grid,),
        in_specs=[
            pl.BlockSpec(memory_space=pltpu.SMEM),
            pl.BlockSpec((_FEAT_BLK, fdim), lambda i: (i, 0)),
            pl.BlockSpec((_FEAT_BLK, fdim), lambda i: (i, 0)),
            pl.BlockSpec((1, fdim), lambda i: (0, 0)),
            pl.BlockSpec((1, fdim), lambda i: (0, 0)),
        ],
        out_specs=pl.BlockSpec((_FEAT_BLK,), lambda i: (i,)),
        out_shape=jax.ShapeDtypeStruct((batch,), jnp.float32),
    )(global_bias, user_features, item_features, w_u, w_i)


def _sc_body(uid_hbm, iid_hbm, uemb_hbm, iemb_hbm, ub_hbm, ib_hbm, feat_hbm,
             out_hbm, uidx_v, iidx_v, urows_v, irows_v, ub_v, ib_v, feat_v,
             out_v, sem):
    wid = lax.axis_index("s") * _NC + lax.axis_index("c")
    base = wid * _BPW
    pltpu.sync_copy(uid_hbm.at[pl.ds(base, _BPW)], uidx_v)
    pltpu.sync_copy(iid_hbm.at[pl.ds(base, _BPW)], iidx_v)
    pltpu.sync_copy(feat_hbm.at[pl.ds(base, _BPW)], feat_v)
    c1 = pltpu.async_copy(uemb_hbm.at[uidx_v], urows_v, sem)
    c2 = pltpu.async_copy(iemb_hbm.at[iidx_v], irows_v, sem)
    c3 = pltpu.async_copy(ub_hbm.at[uidx_v], ub_v, sem)
    c4 = pltpu.async_copy(ib_hbm.at[iidx_v], ib_v, sem)
    c1.wait()
    c2.wait()
    c3.wait()
    c4.wait()

    def group(g, carry):
        rows = g * _L + lax.iota(jnp.int32, _L)
        acc = feat_v[pl.ds(g * _L, _L)]
        acc = acc + ub_v[pl.ds(g * _L, _L)]
        acc = acc + ib_v[pl.ds(g * _L, _L)]
        for k in range(_N_FACTORS):
            cols = jnp.full((_L,), k, jnp.int32)
            u = plsc.load_gather(urows_v, [rows, cols])
            i = plsc.load_gather(irows_v, [rows, cols])
            acc = acc + u * i
        out_v[pl.ds(g * _L, _L)] = acc
        return carry

    lax.fori_loop(0, _GROUPS, group, 0)
    pltpu.sync_copy(out_v, out_hbm.at[pl.ds(base, _BPW)])


def _sc_call(user_ids, item_ids, user_embed, item_embed, user_bias,
             item_bias, feat):
    mesh = plsc.VectorSubcoreMesh(
        core_axis_name="c", subcore_axis_name="s",
        num_cores=_NC, num_subcores=_NS)
    run = pl.kernel(
        _sc_body,
        out_type=jax.ShapeDtypeStruct((_BATCH,), jnp.float32),
        mesh=mesh,
        compiler_params=pltpu.CompilerParams(
            needs_layout_passes=False, use_tc_tiling_on_sc=False),
        scratch_types=[
            pltpu.VMEM((_BPW,), jnp.int32),
            pltpu.VMEM((_BPW,), jnp.int32),
            pltpu.VMEM((_BPW, _N_FACTORS), jnp.float32),
            pltpu.VMEM((_BPW, _N_FACTORS), jnp.float32),
            pltpu.VMEM((_BPW,), jnp.float32),
            pltpu.VMEM((_BPW,), jnp.float32),
            pltpu.VMEM((_BPW,), jnp.float32),
            pltpu.VMEM((_BPW,), jnp.float32),
            pltpu.SemaphoreType.DMA,
        ],
    )
    return run(user_ids, item_ids, user_embed, item_embed, user_bias,
               item_bias, feat)


def kernel(user_ids, item_ids, user_features, item_features, user_embed,
           item_embed, user_bias, item_bias, global_bias, w_u, w_i):
    feat = _feat_call(global_bias, user_features, item_features, w_u, w_i)
    return _sc_call(user_ids, item_ids, user_embed, item_embed,
                    user_bias.reshape(-1), item_bias.reshape(-1), feat)


# speedup vs baseline: 1.4079x; 1.4079x over previous
import functools

import jax
import jax.numpy as jnp
from jax import lax
from jax.experimental import pallas as pl
from jax.experimental.pallas import tpu as pltpu
from jax.experimental.pallas import tpu_sc as plsc

_N_FACTORS = 64
_BATCH = 16384
_NC, _NS, _L = 2, 16, 16
_NW = _NC * _NS
_BPW = _BATCH // _NW
_WAVE = 256
_FEAT_BLK = 2048


def _feat_body(gb_ref, uf_ref, if_ref, wu_ref, wi_ref, out_ref):
    acc = jnp.sum(uf_ref[...] * wu_ref[...], axis=1)
    acc = acc + jnp.sum(if_ref[...] * wi_ref[...], axis=1)
    out_ref[...] = acc + gb_ref[0]


def _feat_call(global_bias, user_features, item_features, w_u, w_i):
    batch, fdim = user_features.shape
    grid = batch // _FEAT_BLK
    return pl.pallas_call(
        _feat_body,
        grid=(grid,),
        in_specs=[
            pl.BlockSpec(memory_space=pltpu.SMEM),
            pl.BlockSpec((_FEAT_BLK, fdim), lambda i: (i, 0)),
            pl.BlockSpec((_FEAT_BLK, fdim), lambda i: (i, 0)),
            pl.BlockSpec((1, fdim), lambda i: (0, 0)),
            pl.BlockSpec((1, fdim), lambda i: (0, 0)),
        ],
        out_specs=pl.BlockSpec((_FEAT_BLK,), lambda i: (i,)),
        out_shape=jax.ShapeDtypeStruct((batch,), jnp.float32),
    )(global_bias, user_features, item_features, w_u, w_i)


def _sc_body(uid_hbm, iid_hbm, uemb_hbm, iemb_hbm, ub_hbm, ib_hbm, feat_hbm,
             out_hbm, uidx_v, iidx_v, urows_v, irows_v,
             ub_v, ib_v, feat_v, out_v, sem, bsem):
    wid = lax.axis_index("s") * _NC + lax.axis_index("c")
    base = wid * _BPW
    pltpu.sync_copy(uid_hbm.at[pl.ds(base, _BPW)], uidx_v)
    pltpu.sync_copy(iid_hbm.at[pl.ds(base, _BPW)], iidx_v)
    pltpu.sync_copy(feat_hbm.at[pl.ds(base, _BPW)], feat_v)
    cb1 = pltpu.async_copy(ub_hbm.at[uidx_v], ub_v, bsem)
    cb2 = pltpu.async_copy(ib_hbm.at[iidx_v], ib_v, bsem)

    cb1.wait()
    cb2.wait()

    def wave(w, carry):
        wrow = w * _WAVE

        def fetch(q, carry2):
            uvec = uidx_v[pl.ds(wrow + q * _L, _L)]
            ivec = iidx_v[pl.ds(wrow + q * _L, _L)]
            for r in range(_L):
                pltpu.async_copy(uemb_hbm.at[uvec[r]],
                                 urows_v.at[q * _L + r, pl.ds(0, _N_FACTORS)],
                                 sem)
                pltpu.async_copy(iemb_hbm.at[ivec[r]],
                                 irows_v.at[q * _L + r, pl.ds(0, _N_FACTORS)],
                                 sem)
            return carry2

        lax.fori_loop(0, _WAVE // _L, fetch, 0)

        def drain(r, carry2):
            pltpu.make_async_copy(
                uemb_hbm.at[0], urows_v.at[r, pl.ds(0, _N_FACTORS)], sem).wait()
            pltpu.make_async_copy(
                iemb_hbm.at[0], irows_v.at[r, pl.ds(0, _N_FACTORS)], sem).wait()
            return carry2

        lax.fori_loop(0, _WAVE, drain, 0)

        def group(g, carry3):
            rows = g * _L + lax.iota(jnp.int32, _L)
            boff = wrow + g * _L
            acc = feat_v[pl.ds(boff, _L)]
            acc = acc + ub_v[pl.ds(boff, _L)]
            acc = acc + ib_v[pl.ds(boff, _L)]
            for k in range(_N_FACTORS):
                cols = jnp.full((_L,), k, jnp.int32)
                u = plsc.load_gather(urows_v, [rows, cols])
                i = plsc.load_gather(irows_v, [rows, cols])
                acc = acc + u * i
            out_v[pl.ds(boff, _L)] = acc
            return carry3

        lax.fori_loop(0, _WAVE // _L, group, 0)
        return carry

    lax.fori_loop(0, _BPW // _WAVE, wave, 0)
    pltpu.sync_copy(out_v, out_hbm.at[pl.ds(base, _BPW)])


def _sc_call(user_ids, item_ids, user_embed, item_embed, user_bias,
             item_bias, feat):
    mesh = plsc.VectorSubcoreMesh(
        core_axis_name="c", subcore_axis_name="s",
        num_cores=_NC, num_subcores=_NS)
    run = pl.kernel(
        _sc_body,
        out_type=jax.ShapeDtypeStruct((_BATCH,), jnp.float32),
        mesh=mesh,
        compiler_params=pltpu.CompilerParams(needs_layout_passes=False),
        scratch_types=[
            pltpu.VMEM((_BPW,), jnp.int32),
            pltpu.VMEM((_BPW,), jnp.int32),
            pltpu.VMEM((_WAVE, _N_FACTORS), jnp.float32),
            pltpu.VMEM((_WAVE, _N_FACTORS), jnp.float32),
            pltpu.VMEM((_BPW,), jnp.float32),
            pltpu.VMEM((_BPW,), jnp.float32),
            pltpu.VMEM((_BPW,), jnp.float32),
            pltpu.VMEM((_BPW,), jnp.float32),
            pltpu.SemaphoreType.DMA,
            pltpu.SemaphoreType.DMA,
        ],
    )
    return run(user_ids, item_ids, user_embed, item_embed, user_bias,
               item_bias, feat)


def kernel(user_ids, item_ids, user_features, item_features, user_embed,
           item_embed, user_bias, item_bias, global_bias, w_u, w_i):
    feat = _feat_call(global_bias, user_features, item_features, w_u, w_i)
    return _sc_call(user_ids, item_ids, user_embed, item_embed,
                    user_bias.reshape(-1), item_bias.reshape(-1), feat)


# speedup vs baseline: 1.8279x; 1.2983x over previous
import functools

import jax
import jax.numpy as jnp
from jax import lax
from jax.experimental import pallas as pl
from jax.experimental.pallas import tpu as pltpu
from jax.experimental.pallas import tpu_sc as plsc

_N_FACTORS = 64
_BATCH = 16384
_N_USERS = 1000000
_NC, _NS, _L = 2, 16, 16
_NW = _NC * _NS
_BPW = _BATCH // _NW
_WAVE = 256
_FEAT_BLK = 2048

_RANGE = 31360
_WBLK = 256
_NWAVE = 124
_SMAX = 999680
_TAIL0 = 999936
_MYCAP = 1024
_VCAP = 256
_NCH = _MYCAP // _L
_SINK = _BATCH


def _feat_body(gb_ref, uf_ref, if_ref, wu_ref, wi_ref, out_ref):
    acc = jnp.sum(uf_ref[...] * wu_ref[...], axis=1)
    acc = acc + jnp.sum(if_ref[...] * wi_ref[...], axis=1)
    out_ref[...] = acc + gb_ref[0]


def _feat_call(global_bias, user_features, item_features, w_u, w_i):
    batch, fdim = user_features.shape
    grid = batch // _FEAT_BLK
    return pl.pallas_call(
        _feat_body,
        grid=(grid,),
        in_specs=[
            pl.BlockSpec(memory_space=pltpu.SMEM),
            pl.BlockSpec((_FEAT_BLK, fdim), lambda i: (i, 0)),
            pl.BlockSpec((_FEAT_BLK, fdim), lambda i: (i, 0)),
            pl.BlockSpec((1, fdim), lambda i: (0, 0)),
            pl.BlockSpec((1, fdim), lambda i: (0, 0)),
        ],
        out_specs=pl.BlockSpec((_FEAT_BLK,), lambda i: (i,)),
        out_shape=jax.ShapeDtypeStruct((batch,), jnp.float32),
    )(global_bias, user_features, item_features, w_u, w_i)


def _extract_body(uid_hbm, uembT_hbm, utail_hbm, urows_hbm, ids_v, myu_v,
                  myb_v, slab_a, slab_b, vrows_v, bidx_v, utail_v, fsem,
                  fsem_b, ssem):
    wid = lax.axis_index("s") * _NC + lax.axis_index("c")
    col_lo = wid * _RANGE
    pltpu.sync_copy(uid_hbm, ids_v)
    pltpu.sync_copy(utail_hbm, utail_v)
    iota = lax.iota(jnp.int32, _L)

    def scan(ch, cnt):
        uvec = ids_v[pl.ds(ch * _L, _L)]
        bvec = ch * _L + iota
        m = (uvec >= col_lo) & (uvec < col_lo + _RANGE)
        rank = plsc.cumsum(m.astype(jnp.int32)) - 1
        rowidx = cnt + rank
        plsc.store_scatter(myu_v, [rowidx], uvec, mask=m)
        plsc.store_scatter(myb_v, [rowidx], bvec, mask=m)
        return cnt + jnp.sum(m.astype(jnp.int32))

    count = lax.fori_loop(0, _BATCH // _L, scan, 0)
    nch = (count + _L - 1) // _L

    def binit(g, c):
        bidx_v[pl.ds(g * _L, _L)] = jnp.full((_L,), _SINK, jnp.int32)
        return c

    lax.fori_loop(0, _VCAP // _L, binit, 0)

    def flush(vcnt):
        pltpu.async_copy(vrows_v, urows_hbm.at[bidx_v], ssem)
        pltpu.make_async_copy(vrows_v, urows_hbm.at[bidx_v], ssem).wait()
        lax.fori_loop(0, _VCAP // _L, binit, 0)
        return 0

    def extract(buf, s, lo, vcnt0):
        def chunk(ch, vcnt):
            uvec = myu_v[pl.ds(ch * _L, _L)]
            bvec = myb_v[pl.ds(ch * _L, _L)]
            valid = (ch * _L + iota) < count
            m = valid & (uvec >= lo) & (uvec < lo + s)
            npc = jnp.sum(m.astype(jnp.int32))

            def hit(vc):
                vc = lax.cond(vc + _L > _VCAP, flush, lambda x: x, vc)
                rank = plsc.cumsum(m.astype(jnp.int32)) - 1
                rowidx = vc + rank
                offs = uvec - lo
                for k in range(_N_FACTORS):
                    kv = jnp.full((_L,), k, jnp.int32)
                    g = plsc.load_gather(buf, [kv, offs], mask=m)
                    plsc.store_scatter(vrows_v, [rowidx, kv], g, mask=m)
                plsc.store_scatter(bidx_v, [rowidx], bvec, mask=m)
                return vc + npc

            return lax.cond(npc > 0, hit, lambda vc: vc, vcnt)

        return lax.fori_loop(0, nch, chunk, vcnt0)

    def wstart(v):
        return pl.multiple_of(
            jnp.minimum(col_lo + _WBLK * v, _SMAX), 128)

    def fetch(v, buf, fs):
        pltpu.async_copy(
            uembT_hbm.at[:, pl.ds(wstart(v), _WBLK)], buf, fs)

    def fwait(buf, fs):
        pltpu.make_async_copy(
            uembT_hbm.at[:, pl.ds(0, _WBLK)], buf, fs).wait()

    fetch(0, slab_a, fsem)

    def pair(v2, vcnt):
        va = 2 * v2
        fwait(slab_a, fsem)
        fetch(va + 1, slab_b, fsem_b)
        vcnt = extract(slab_a, _WBLK, wstart(va), vcnt)
        fwait(slab_b, fsem_b)

        @pl.when(v2 < _NWAVE // 2 - 1)
        def _():
            fetch(va + 2, slab_a, fsem)

        vcnt = extract(slab_b, _WBLK, wstart(va + 1), vcnt)
        return vcnt

    vcnt = lax.fori_loop(0, _NWAVE // 2, pair, 0)
    vcnt = extract(utail_v, _N_USERS - _TAIL0, _TAIL0, vcnt)
    flush(vcnt)


def _extract_call(user_ids, uembT, utail):
    mesh = plsc.VectorSubcoreMesh(
        core_axis_name="c", subcore_axis_name="s",
        num_cores=_NC, num_subcores=_NS)
    run = pl.kernel(
        _extract_body,
        out_type=jax.ShapeDtypeStruct((_BATCH + _L, 128), jnp.float32),
        mesh=mesh,
        compiler_params=pltpu.CompilerParams(needs_layout_passes=False),
        scratch_types=[
            pltpu.VMEM((_BATCH,), jnp.int32),
            pltpu.VMEM((_MYCAP,), jnp.int32),
            pltpu.VMEM((_MYCAP,), jnp.int32),
            pltpu.VMEM((_N_FACTORS, _WBLK), jnp.float32),
            pltpu.VMEM((_N_FACTORS, _WBLK), jnp.float32),
            pltpu.VMEM((_VCAP, 128), jnp.float32),
            pltpu.VMEM((_VCAP,), jnp.int32),
            pltpu.VMEM((_N_FACTORS, _N_USERS - _TAIL0), jnp.float32),
            pltpu.SemaphoreType.DMA,
            pltpu.SemaphoreType.DMA,
            pltpu.SemaphoreType.DMA,
        ],
    )
    return run(user_ids, uembT, utail)


def _sc_body(uid_hbm, iid_hbm, urows_hbm, iemb_hbm, ub_hbm, ib_hbm, feat_hbm,
             out_hbm, uidx_v, iidx_v, ucols_v, icols_v,
             ub_v, ib_v, feat_v, out_v, sem, bsem):
    wid = lax.axis_index("s") * _NC + lax.axis_index("c")
    base = wid * _BPW
    pltpu.sync_copy(uid_hbm.at[pl.ds(base, _BPW)], uidx_v)
    pltpu.sync_copy(iid_hbm.at[pl.ds(base, _BPW)], iidx_v)
    pltpu.sync_copy(feat_hbm.at[pl.ds(base, _BPW)], feat_v)
    cb1 = pltpu.async_copy(ub_hbm.at[uidx_v], ub_v, bsem)
    cb2 = pltpu.async_copy(ib_hbm.at[iidx_v], ib_v, bsem)

    cb1.wait()
    cb2.wait()

    def wave(w, carry):
        wrow = w * _WAVE
        pltpu.async_copy(
            urows_hbm.at[pl.ds(base + wrow, _WAVE)], ucols_v, bsem)

        def fetch(q, carry2):
            ivec = iidx_v[pl.ds(wrow + q * _L, _L)]
            for r in range(_L):
                c = q * _L + r
                pltpu.async_copy(iemb_hbm.at[ivec[r]],
                                 icols_v.at[c, pl.ds(0, _N_FACTORS)], sem)
            return carry2

        lax.fori_loop(0, _WAVE // _L, fetch, 0)

        def drain(g, carry2):
            pltpu.make_async_copy(
                iemb_hbm.at[0],
                icols_v.at[0, pl.ds(0, _N_FACTORS)], sem).wait()
            return carry2

        lax.fori_loop(0, _WAVE, drain, 0)
        pltpu.make_async_copy(
            urows_hbm.at[pl.ds(0, _WAVE)], ucols_v, bsem).wait()

        def group(g, carry3):
            boff = g * _L
            rows = boff + lax.iota(jnp.int32, _L)
            acc = feat_v[pl.ds(wrow + boff, _L)]
            acc = acc + ub_v[pl.ds(wrow + boff, _L)]
            acc = acc + ib_v[pl.ds(wrow + boff, _L)]
            for k in range(_N_FACTORS):
                cols = jnp.full((_L,), k, jnp.int32)
                u = plsc.load_gather(ucols_v, [rows, cols])
                i = plsc.load_gather(icols_v, [rows, cols])
                acc = acc + u * i
            out_v[pl.ds(wrow + boff, _L)] = acc
            return carry3

        lax.fori_loop(0, _WAVE // _L, group, 0)
        return carry

    lax.fori_loop(0, _BPW // _WAVE, wave, 0)
    pltpu.sync_copy(out_v, out_hbm.at[pl.ds(base, _BPW)])


def _sc_call(user_ids, item_ids, urows, item_embed, user_bias,
             item_bias, feat):
    mesh = plsc.VectorSubcoreMesh(
        core_axis_name="c", subcore_axis_name="s",
        num_cores=_NC, num_subcores=_NS)
    run = pl.kernel(
        _sc_body,
        out_type=jax.ShapeDtypeStruct((_BATCH,), jnp.float32),
        mesh=mesh,
        compiler_params=pltpu.CompilerParams(needs_layout_passes=False),
        scratch_types=[
            pltpu.VMEM((_BPW,), jnp.int32),
            pltpu.VMEM((_BPW,), jnp.int32),
            pltpu.VMEM((_WAVE, 128), jnp.float32),
            pltpu.VMEM((_WAVE, _N_FACTORS), jnp.float32),
            pltpu.VMEM((_BPW,), jnp.float32),
            pltpu.VMEM((_BPW,), jnp.float32),
            pltpu.VMEM((_BPW,), jnp.float32),
            pltpu.VMEM((_BPW,), jnp.float32),
            pltpu.SemaphoreType.DMA,
            pltpu.SemaphoreType.DMA,
        ],
    )
    return run(user_ids, item_ids, urows, item_embed, user_bias,
               item_bias, feat)


def kernel(user_ids, item_ids, user_features, item_features, user_embed,
           item_embed, user_bias, item_bias, global_bias, w_u, w_i):
    feat = _feat_call(global_bias, user_features, item_features, w_u, w_i)
    utail = user_embed[_TAIL0:, :].T
    urows = _extract_call(user_ids, user_embed.T, utail)
    return _sc_call(user_ids, item_ids, urows, item_embed,
                    user_bias.reshape(-1), item_bias.reshape(-1), feat)
